# baseline (speedup 1.0000x reference)
import functools

import jax
import jax.numpy as jnp
from jax import lax
from jax.experimental import pallas as pl
from jax.experimental.pallas import tpu as pltpu
from jax.experimental.pallas import tpu_sc as plsc

NUM_COLS = 13
CAT_COLS = 26
VOCAB = 100
EMB_DIM = 64
BATCH = 16384

NC = 2
NS = 16
LANES = 16
NW = NC * NS
BPW = BATCH // NW
IDX_MINOR = 128
HALF = BPW // 2
QTR = BPW // 4
STG = 168
N_ITEMS = 4 * (CAT_COLS // 2)

_mesh = plsc.VectorSubcoreMesh(core_axis_name="c", subcore_axis_name="s")


def _nums_body(*refs):
    o_ref = refs[-1]
    o_ref[...] = jnp.stack([r[...] for r in refs[:-1]], axis=1)


_NB = 1024
_nums_stack = pl.pallas_call(
    _nums_body,
    grid=(BATCH // _NB,),
    in_specs=[pl.BlockSpec((_NB,), lambda i: (i,))] * NUM_COLS,
    out_specs=pl.BlockSpec((_NB, NUM_COLS), lambda i: (i, 0)),
    out_shape=jax.ShapeDtypeStruct((BATCH, NUM_COLS), jnp.float32),
)


@functools.partial(
    pl.kernel,
    mesh=_mesh,
    out_type=jax.ShapeDtypeStruct((BATCH, CAT_COLS * EMB_DIM), jnp.float32),
    scratch_types=[
        pltpu.VMEM_SHARED((CAT_COLS * VOCAB, 2 * EMB_DIM), jnp.float32),
        pltpu.VMEM_SHARED((CAT_COLS * VOCAB, 2 * EMB_DIM), jnp.float32),
        pltpu.VMEM((2, HALF), jnp.int32),
        pltpu.VMEM((2, HALF), jnp.int32),
        pltpu.VMEM((QTR, 2 * EMB_DIM), jnp.float32),
        pltpu.VMEM((QTR, 2 * EMB_DIM), jnp.float32),
        pltpu.VMEM((QTR, 2 * EMB_DIM), jnp.float32),
        pltpu.VMEM((QTR, 2 * EMB_DIM), jnp.float32),
        pltpu.SemaphoreType.DMA((4,)),
        pltpu.SemaphoreType.DMA((4,)),
    ],
)
def _cat_gather(tl_hbm, tr_hbm, *rest):
    cats = rest[:CAT_COLS]
    (out_hbm, tl_sh, tr_sh, idxa_v, idxb_v,
     slab0, slab1, slab2, slab3, semg, semw) = rest[CAT_COLS:]
    c = lax.axis_index("c")
    s = lax.axis_index("s")
    wid = s * NC + c
    b0 = wid * BPW
    slabs = (slab0, slab1, slab2, slab3)

    @pl.when(s < NS - 1)
    def _stage_main():
        for src, dst in ((tl_hbm, tl_sh), (tr_hbm, tr_sh)):
            pltpu.sync_copy(src.at[pl.ds(s * STG, STG)],
                            dst.at[pl.ds(s * STG, STG)])

    @pl.when(s == NS - 1)
    def _stage_tail():
        for src, dst in ((tl_hbm, tl_sh), (tr_hbm, tr_sh)):
            pltpu.sync_copy(src.at[pl.ds((NS - 1) * STG, 80)],
                            dst.at[pl.ds((NS - 1) * STG, 80)])

    plsc.subcore_barrier()


    def tbl(base, col):
        return base.at[pl.ds(col * VOCAB, VOCAB)]

    def prep(i):
        sid = q = i % 4
        h, rr = divmod(q, 2)
        j = i // 4

        if i >= 4:
            pltpu.make_async_copy(
                slabs[sid],
                out_hbm.at[pl.ds(b0, QTR), pl.ds(0, 2 * EMB_DIM)],
                semw.at[sid],
            ).wait()

        if rr == 0:
            pltpu.sync_copy(cats[2 * j].at[pl.ds(b0 + h * HALF, HALF)],
                            idxa_v.at[h])
            pltpu.sync_copy(cats[2 * j + 1].at[pl.ds(b0 + h * HALF, HALF)],
                            idxb_v.at[h])
        pltpu.async_copy(
            tbl(tl_sh, 2 * j).at[idxa_v.at[h, pl.ds(rr * QTR, QTR)]],
            slabs[sid],
            semg.at[sid],
        )

    def fire_b(i):
        sid = q = i % 4
        h, rr = divmod(q, 2)
        j = i // 4
        pltpu.make_async_copy(
            tbl(tl_sh, 2 * j).at[idxa_v.at[h, pl.ds(rr * QTR, QTR)]],
            slabs[sid],
            semg.at[sid],
        ).wait()
        pltpu.async_copy(
            tbl(tr_sh, 2 * j + 1).at[idxb_v.at[h, pl.ds(rr * QTR, QTR)]],
            slabs[sid],
            semg.at[sid],
            add=True,
        )

    def finish_w(i):
        sid = q = i % 4
        h, rr = divmod(q, 2)
        j = i // 4
        pltpu.make_async_copy(
            tbl(tr_sh, 2 * j + 1).at[idxb_v.at[h, pl.ds(rr * QTR, QTR)]],
            slabs[sid],
            semg.at[sid],
        ).wait()
        pltpu.async_copy(
            slabs[sid],
            out_hbm.at[pl.ds(b0 + q * QTR, QTR),
                       pl.ds(j * 2 * EMB_DIM, 2 * EMB_DIM)],
            semw.at[sid],
        )

    prep(0)
    prep(1)
    for t in range(N_ITEMS):
        fire_b(t)
        if t >= 1:
            finish_w(t - 1)
        if t + 2 < N_ITEMS:
            prep(t + 2)
    finish_w(N_ITEMS - 1)
    for sid in range(4):
        pltpu.make_async_copy(
            slabs[sid],
            out_hbm.at[pl.ds(b0, QTR), pl.ds(0, 2 * EMB_DIM)],
            semw.at[sid],
        ).wait()


def kernel(num_0, num_1, num_2, num_3, num_4, num_5, num_6, num_7, num_8, num_9, num_10, num_11, num_12, cat_0, cat_1, cat_2, cat_3, cat_4, cat_5, cat_6, cat_7, cat_8, cat_9, cat_10, cat_11, cat_12, cat_13, cat_14, cat_15, cat_16, cat_17, cat_18, cat_19, cat_20, cat_21, cat_22, cat_23, cat_24, cat_25, emb_0, emb_1, emb_2, emb_3, emb_4, emb_5, emb_6, emb_7, emb_8, emb_9, emb_10, emb_11, emb_12, emb_13, emb_14, emb_15, emb_16, emb_17, emb_18, emb_19, emb_20, emb_21, emb_22, emb_23, emb_24, emb_25):
    nums = [num_0, num_1, num_2, num_3, num_4, num_5, num_6, num_7, num_8,
            num_9, num_10, num_11, num_12]
    cats = [cat_0, cat_1, cat_2, cat_3, cat_4, cat_5, cat_6, cat_7, cat_8,
            cat_9, cat_10, cat_11, cat_12, cat_13, cat_14, cat_15, cat_16,
            cat_17, cat_18, cat_19, cat_20, cat_21, cat_22, cat_23, cat_24,
            cat_25]
    embs = [emb_0, emb_1, emb_2, emb_3, emb_4, emb_5, emb_6, emb_7, emb_8,
            emb_9, emb_10, emb_11, emb_12, emb_13, emb_14, emb_15, emb_16,
            emb_17, emb_18, emb_19, emb_20, emb_21, emb_22, emb_23, emb_24,
            emb_25]

    table = jnp.concatenate(embs, axis=0)
    zeros = jnp.zeros_like(table)
    table_l = jnp.concatenate([table, zeros], axis=1)
    table_r = jnp.concatenate([zeros, table], axis=1)
    x_cats = _cat_gather(table_l, table_r, *cats)

    x_nums = _nums_stack(*nums)
    return (x_nums, x_cats)
